# Initial kernel scaffold; baseline (speedup 1.0000x reference)
import jax
import jax.numpy as jnp
from jax.experimental import pallas as pl


def kernel(x, edge_index, origin_edge_features, W_ef, b_ef, W1, b1, W2, b2, W3, b3):
    raise NotImplementedError("write your pallas kernel here")


# speedup vs baseline: 2.5257x; 2.5257x over previous
import functools

import jax
import jax.numpy as jnp
from jax import lax
from jax.experimental import pallas as pl
from jax.experimental.pallas import tpu as pltpu
from jax.experimental.pallas import tpu_sc as plsc

_N = 10000
_D = 128
_E = 320000
_DE = 16
_P = 128

_NC = 2
_NS = 16
_NW = _NC * _NS
_B = 128
_BT = (-(-_E // (_NW * _B)) + 7) // 8 * 8
_EPAD = _NW * _BT * _B
_ACC = 10240
_CPT = _ACC // _NS // _B
_R0 = _ACC // _NS

_mesh = plsc.VectorSubcoreMesh(
    core_axis_name="c", subcore_axis_name="s", num_cores=_NC, num_subcores=_NS)


def _make_segsum(width):
    ch = width // 16

    def body(table_hbm, src_hbm, dst_hbm, out_hbm, src_v, dst_v, rows_v, acc, sem):
        cid = lax.axis_index("c")
        sid = lax.axis_index("s")
        wid = sid * _NC + cid

        def zrow(i, c):
            rows_v[i // ch, pl.ds((i % ch) * 16, 16)] = jnp.zeros((16,), jnp.float32)
            return c
        lax.fori_loop(0, _B * ch, zrow, 0)

        row0 = sid * _R0

        def zacc(k, c):
            pltpu.sync_copy(rows_v, acc.at[pl.ds(row0 + k * _B, _B)])
            return c
        lax.fori_loop(0, _CPT, zacc, 0)
        plsc.subcore_barrier()

        pltpu.sync_copy(src_hbm.at[wid], src_v)
        pltpu.sync_copy(dst_hbm.at[wid], dst_v)

        def edge_batch(j, c):
            pltpu.async_copy(table_hbm.at[src_v.at[j]], rows_v, sem).wait()
            pltpu.sync_copy(rows_v, acc.at[dst_v.at[j]], add=True)
            return c
        lax.fori_loop(0, _BT, edge_batch, 0)
        plsc.subcore_barrier()

        def copyout(k, c):
            pltpu.sync_copy(acc.at[pl.ds(row0 + k * _B, _B)], rows_v)
            pltpu.sync_copy(rows_v, out_hbm.at[cid, pl.ds(row0 + k * _B, _B)])
            return c
        lax.fori_loop(0, _CPT, copyout, 0)

    return functools.partial(
        pl.kernel,
        out_type=jax.ShapeDtypeStruct((_NC, _ACC, width), jnp.float32),
        mesh=_mesh,
        scratch_types=[
            pltpu.VMEM((_BT, _B), jnp.int32),
            pltpu.VMEM((_BT, _B), jnp.int32),
            pltpu.VMEM((_B, width), jnp.float32),
            pltpu.VMEM_SHARED((_ACC, width), jnp.float32),
            pltpu.SemaphoreType.DMA,
        ],
    )(body)


_sc_seg_d = _make_segsum(_D)
_sc_seg_p = _sc_seg_d


_TCR = 1000


def _tc_layer_body(p_ref, pk_ref, wh_ref, u_ref, o_ref):
    p = p_ref[0] + p_ref[1]
    pk = pk_ref[0] + pk_ref[1]
    cnt = pk[:, _DE:_DE + 1]
    invc = 1.0 / jnp.maximum(cnt, 1.0)
    h = jnp.dot(p * invc, wh_ref[...], preferred_element_type=jnp.float32)
    h = h + jnp.dot(pk * invc, u_ref[...], preferred_element_type=jnp.float32)
    o_ref[...] = jnp.maximum(h, 0.0)


_tc_layer = pl.pallas_call(
    _tc_layer_body,
    grid=(_N // _TCR,),
    in_specs=[
        pl.BlockSpec((_NC, _TCR, _D), lambda m: (0, m, 0)),
        pl.BlockSpec((_NC, _TCR, _P), lambda m: (0, m, 0)),
        pl.BlockSpec((_D, _D), lambda m: (0, 0)),
        pl.BlockSpec((_P, _D), lambda m: (0, 0)),
    ],
    out_specs=pl.BlockSpec((_TCR, _D), lambda m: (m, 0)),
    out_shape=jax.ShapeDtypeStruct((_N, _D), jnp.float32),
)


def kernel(x, edge_index, origin_edge_features, W_ef, b_ef, W1, b1, W2, b2, W3, b3):
    src = edge_index[0].astype(jnp.int32)
    dst = edge_index[1].astype(jnp.int32)
    pad = _EPAD - _E
    src_p = jnp.concatenate(
        [src, jnp.zeros((pad,), jnp.int32)]).reshape(_NW, _BT, _B)
    dst_p = jnp.concatenate(
        [dst, jnp.full((pad,), _N, jnp.int32)]).reshape(_NW, _BT, _B)

    packed = jnp.concatenate(
        [origin_edge_features,
         jnp.ones((_E, 1), jnp.float32),
         jnp.zeros((_E, _P - _DE - 1), jnp.float32)], axis=1)
    packed = jnp.concatenate([packed, jnp.zeros((pad, _P), jnp.float32)], axis=0)
    lin = jnp.arange(_EPAD, dtype=jnp.int32).reshape(_NW, _BT, _B)

    pk = _sc_seg_p(packed, lin, dst_p)

    h = x
    for W, b in ((W1, b1), (W2, b2), (W3, b3)):
        Wh, We = W[:_D], W[_D:]
        U = jnp.concatenate(
            [W_ef @ We,
             (b_ef @ We + b).reshape(1, _D),
             jnp.zeros((_P - _DE - 1, _D), jnp.float32)], axis=0)
        p = _sc_seg_d(h, src_p, dst_p)
        h = _tc_layer(p, pk, Wh, U)
    return h


# speedup vs baseline: 2.5744x; 1.0193x over previous
import functools

import jax
import jax.numpy as jnp
from jax import lax
from jax.experimental import pallas as pl
from jax.experimental.pallas import tpu as pltpu
from jax.experimental.pallas import tpu_sc as plsc

_N = 10000
_D = 128
_E = 320000
_DE = 16
_P = 128

_NC = 2
_NS = 16
_NW = _NC * _NS
_B = 128
_BT = (-(-_E // (_NW * _B)) + 7) // 8 * 8
_EPAD = _NW * _BT * _B
_ACC = 10240
_CPT = _ACC // _NS // _B
_R0 = _ACC // _NS

_mesh = plsc.VectorSubcoreMesh(
    core_axis_name="c", subcore_axis_name="s", num_cores=_NC, num_subcores=_NS)


_K = 2
_HB = _BT // 2


def _make_segsum(width):
    ch = width // 16

    def body(table_hbm, src_hbm, dst_hbm, out_hbm, src_v, dst_v, rows_v, acc, sem):
        cid = lax.axis_index("c")
        sid = lax.axis_index("s")
        wid = sid * _NC + cid

        def zrow(i, c):
            rows_v[i // ch, pl.ds((i % ch) * 16, 16)] = jnp.zeros((16,), jnp.float32)
            return c
        lax.fori_loop(0, _B * ch, zrow, 0)

        row0 = sid * _R0

        def zacc(k, c):
            pltpu.sync_copy(rows_v.at[pl.ds(0, _B)], acc.at[pl.ds(row0 + k * _B, _B)])
            return c
        lax.fori_loop(0, _CPT, zacc, 0)
        plsc.subcore_barrier()

        def half_loop(hf, c):
            pltpu.sync_copy(src_hbm.at[wid, pl.ds(hf * _HB, _HB)], src_v)
            pltpu.sync_copy(dst_hbm.at[wid, pl.ds(hf * _HB, _HB)], dst_v)

            def edge_group(g, c2):
                handles = []
                for i in range(_K):
                    handles.append(pltpu.async_copy(
                        table_hbm.at[src_v.at[g * _K + i]],
                        rows_v.at[pl.ds(i * _B, _B)], sem))
                for hd in handles:
                    hd.wait()
                for i in range(_K):
                    pltpu.sync_copy(rows_v.at[pl.ds(i * _B, _B)],
                                    acc.at[dst_v.at[g * _K + i]], add=True)
                return c2
            lax.fori_loop(0, _HB // _K, edge_group, 0)
            return c
        lax.fori_loop(0, 2, half_loop, 0)
        plsc.subcore_barrier()

        def copyout(k, c):
            pltpu.sync_copy(acc.at[pl.ds(row0 + k * _B, _B)], rows_v.at[pl.ds(0, _B)])
            pltpu.sync_copy(rows_v.at[pl.ds(0, _B)],
                            out_hbm.at[cid, pl.ds(row0 + k * _B, _B)])
            return c
        lax.fori_loop(0, _CPT, copyout, 0)

    return functools.partial(
        pl.kernel,
        out_type=jax.ShapeDtypeStruct((_NC, _ACC, width), jnp.float32),
        mesh=_mesh,
        scratch_types=[
            pltpu.VMEM((_HB, _B), jnp.int32),
            pltpu.VMEM((_HB, _B), jnp.int32),
            pltpu.VMEM((_K * _B, width), jnp.float32),
            pltpu.VMEM_SHARED((_ACC, width), jnp.float32),
            pltpu.SemaphoreType.DMA,
        ],
    )(body)


_sc_seg_d = _make_segsum(_D)
_sc_seg_p = _sc_seg_d


_TCR = 1000


def _tc_layer_body(p_ref, pk_ref, wh_ref, u_ref, o_ref):
    p = p_ref[0] + p_ref[1]
    pk = pk_ref[0] + pk_ref[1]
    cnt = pk[:, _DE:_DE + 1]
    invc = 1.0 / jnp.maximum(cnt, 1.0)
    h = jnp.dot(p * invc, wh_ref[...], preferred_element_type=jnp.float32)
    h = h + jnp.dot(pk * invc, u_ref[...], preferred_element_type=jnp.float32)
    o_ref[...] = jnp.maximum(h, 0.0)


_tc_layer = pl.pallas_call(
    _tc_layer_body,
    grid=(_N // _TCR,),
    in_specs=[
        pl.BlockSpec((_NC, _TCR, _D), lambda m: (0, m, 0)),
        pl.BlockSpec((_NC, _TCR, _P), lambda m: (0, m, 0)),
        pl.BlockSpec((_D, _D), lambda m: (0, 0)),
        pl.BlockSpec((_P, _D), lambda m: (0, 0)),
    ],
    out_specs=pl.BlockSpec((_TCR, _D), lambda m: (m, 0)),
    out_shape=jax.ShapeDtypeStruct((_N, _D), jnp.float32),
)


def kernel(x, edge_index, origin_edge_features, W_ef, b_ef, W1, b1, W2, b2, W3, b3):
    src = edge_index[0].astype(jnp.int32)
    dst = edge_index[1].astype(jnp.int32)
    pad = _EPAD - _E
    src_p = jnp.concatenate(
        [src, jnp.zeros((pad,), jnp.int32)]).reshape(_NW, _BT, _B)
    dst_p = jnp.concatenate(
        [dst, jnp.full((pad,), _N, jnp.int32)]).reshape(_NW, _BT, _B)

    packed = jnp.concatenate(
        [origin_edge_features,
         jnp.ones((_E, 1), jnp.float32),
         jnp.zeros((_E, _P - _DE - 1), jnp.float32)], axis=1)
    packed = jnp.concatenate([packed, jnp.zeros((pad, _P), jnp.float32)], axis=0)
    lin = jnp.arange(_EPAD, dtype=jnp.int32).reshape(_NW, _BT, _B)

    pk = _sc_seg_p(packed, lin, dst_p)

    h = x
    for W, b in ((W1, b1), (W2, b2), (W3, b3)):
        Wh, We = W[:_D], W[_D:]
        U = jnp.concatenate(
            [W_ef @ We,
             (b_ef @ We + b).reshape(1, _D),
             jnp.zeros((_P - _DE - 1, _D), jnp.float32)], axis=0)
        p = _sc_seg_d(h, src_p, dst_p)
        h = _tc_layer(p, pk, Wh, U)
    return h


# speedup vs baseline: 2.6451x; 1.0274x over previous
import functools

import jax
import jax.numpy as jnp
from jax import lax
from jax.experimental import pallas as pl
from jax.experimental.pallas import tpu as pltpu
from jax.experimental.pallas import tpu_sc as plsc

_N = 10000
_D = 128
_E = 320000
_DE = 16
_P = 128

_NC = 2
_NS = 16
_NW = _NC * _NS
_B = 128
_BT = (-(-_E // (_NW * _B)) + 7) // 8 * 8
_EPAD = _NW * _BT * _B
_ACC = 10240
_CPT = _ACC // _NS // _B
_R0 = _ACC // _NS

_mesh = plsc.VectorSubcoreMesh(
    core_axis_name="c", subcore_axis_name="s", num_cores=_NC, num_subcores=_NS)


_K = 2
_HB = _BT // 2


def _make_segsum(width):
    ch = width // 16

    def body(table_hbm, src_hbm, dst_hbm, out_hbm, src_v, dst_v, rows_v, acc, sem):
        cid = lax.axis_index("c")
        sid = lax.axis_index("s")
        wid = sid * _NC + cid

        def zrow(i, c):
            rows_v[i // ch, pl.ds((i % ch) * 16, 16)] = jnp.zeros((16,), jnp.float32)
            return c
        lax.fori_loop(0, _B * ch, zrow, 0)

        row0 = sid * _R0

        def zacc(k, c):
            pltpu.sync_copy(rows_v.at[pl.ds(0, _B)], acc.at[pl.ds(row0 + k * _B, _B)])
            return c
        lax.fori_loop(0, _CPT, zacc, 0)
        plsc.subcore_barrier()

        def half_loop(hf, c):
            pltpu.sync_copy(src_hbm.at[wid, pl.ds(hf * _HB, _HB)], src_v)
            pltpu.sync_copy(dst_hbm.at[wid, pl.ds(hf * _HB, _HB)], dst_v)

            def edge_group(g, c2):
                handles = []
                for i in range(_K):
                    handles.append(pltpu.async_copy(
                        table_hbm.at[src_v.at[g * _K + i]],
                        rows_v.at[pl.ds(i * _B, _B)], sem))
                for i in range(_K):
                    handles[i].wait()
                    pltpu.sync_copy(rows_v.at[pl.ds(i * _B, _B)],
                                    acc.at[dst_v.at[g * _K + i]], add=True)
                return c2
            lax.fori_loop(0, _HB // _K, edge_group, 0)
            return c
        lax.fori_loop(0, 2, half_loop, 0)
        plsc.subcore_barrier()

        def copyout(k, c):
            pltpu.sync_copy(acc.at[pl.ds(row0 + k * _B, _B)], rows_v.at[pl.ds(0, _B)])
            pltpu.sync_copy(rows_v.at[pl.ds(0, _B)],
                            out_hbm.at[cid, pl.ds(row0 + k * _B, _B)])
            return c
        lax.fori_loop(0, _CPT, copyout, 0)

    return functools.partial(
        pl.kernel,
        out_type=jax.ShapeDtypeStruct((_NC, _ACC, width), jnp.float32),
        mesh=_mesh,
        scratch_types=[
            pltpu.VMEM((_HB, _B), jnp.int32),
            pltpu.VMEM((_HB, _B), jnp.int32),
            pltpu.VMEM((_K * _B, width), jnp.float32),
            pltpu.VMEM_SHARED((_ACC, width), jnp.float32),
            pltpu.SemaphoreType.DMA,
        ],
    )(body)


_sc_seg_d = _make_segsum(_D)
_sc_seg_p = _sc_seg_d


_TCR = 1000


def _tc_layer_body(p_ref, pk_ref, wh_ref, u_ref, o_ref):
    p = p_ref[0] + p_ref[1]
    pk = pk_ref[0] + pk_ref[1]
    cnt = pk[:, _DE:_DE + 1]
    invc = 1.0 / jnp.maximum(cnt, 1.0)
    h = jnp.dot(p * invc, wh_ref[...], preferred_element_type=jnp.float32)
    h = h + jnp.dot(pk * invc, u_ref[...], preferred_element_type=jnp.float32)
    o_ref[...] = jnp.maximum(h, 0.0)


_tc_layer = pl.pallas_call(
    _tc_layer_body,
    grid=(_N // _TCR,),
    in_specs=[
        pl.BlockSpec((_NC, _TCR, _D), lambda m: (0, m, 0)),
        pl.BlockSpec((_NC, _TCR, _P), lambda m: (0, m, 0)),
        pl.BlockSpec((_D, _D), lambda m: (0, 0)),
        pl.BlockSpec((_P, _D), lambda m: (0, 0)),
    ],
    out_specs=pl.BlockSpec((_TCR, _D), lambda m: (m, 0)),
    out_shape=jax.ShapeDtypeStruct((_N, _D), jnp.float32),
)


def kernel(x, edge_index, origin_edge_features, W_ef, b_ef, W1, b1, W2, b2, W3, b3):
    src = edge_index[0].astype(jnp.int32)
    dst = edge_index[1].astype(jnp.int32)
    pad = _EPAD - _E
    src_p = jnp.concatenate(
        [src, jnp.zeros((pad,), jnp.int32)]).reshape(_NW, _BT, _B)
    dst_p = jnp.concatenate(
        [dst, jnp.full((pad,), _N, jnp.int32)]).reshape(_NW, _BT, _B)

    packed = jnp.concatenate(
        [origin_edge_features,
         jnp.ones((_E, 1), jnp.float32),
         jnp.zeros((_E, _P - _DE - 1), jnp.float32)], axis=1)
    packed = jnp.concatenate([packed, jnp.zeros((pad, _P), jnp.float32)], axis=0)
    lin = jnp.arange(_EPAD, dtype=jnp.int32).reshape(_NW, _BT, _B)

    pk = _sc_seg_p(packed, lin, dst_p)

    h = x
    for W, b in ((W1, b1), (W2, b2), (W3, b3)):
        Wh, We = W[:_D], W[_D:]
        U = jnp.concatenate(
            [W_ef @ We,
             (b_ef @ We + b).reshape(1, _D),
             jnp.zeros((_P - _DE - 1, _D), jnp.float32)], axis=0)
        p = _sc_seg_d(h, src_p, dst_p)
        h = _tc_layer(p, pk, Wh, U)
    return h
